# baseline (speedup 1.0000x reference)
import functools

import jax
import jax.numpy as jnp
from jax import lax
from jax.experimental import pallas as pl
from jax.experimental.pallas import tpu as pltpu
from jax.experimental.pallas import tpu_sc as plsc

_NC = 2
_NS = 16
_NW = _NC * _NS
_L = 16
_CHUNK = 320
_SUB = 128


def _mesh():
    return plsc.VectorSubcoreMesh(
        core_axis_name="c", subcore_axis_name="s", num_cores=_NC,
        num_subcores=_NS)


def _enc_body(x_ref, w1_ref, b1_ref, w2_ref, b2_ref, h_ref):
    h = jnp.maximum(
        jnp.dot(x_ref[...].astype(jnp.bfloat16), w1_ref[...],
                preferred_element_type=jnp.float32) + b1_ref[...], 0.0)
    h = jnp.maximum(
        jnp.dot(h.astype(jnp.bfloat16), w2_ref[...],
                preferred_element_type=jnp.float32) + b2_ref[...], 0.0)
    h_ref[...] = h


def _tc_encode(x, w1, b1, w2, b2):
    n, din = x.shape
    hdim = w1.shape[1]
    bn = 2048
    grid = n // bn
    const = lambda i: (0, 0)
    return pl.pallas_call(
        _enc_body,
        grid=(grid,),
        in_specs=[
            pl.BlockSpec((bn, din), lambda i: (i, 0)),
            pl.BlockSpec((din, hdim), const),
            pl.BlockSpec((1, hdim), const),
            pl.BlockSpec((hdim, hdim), const),
            pl.BlockSpec((1, hdim), const),
        ],
        out_specs=pl.BlockSpec((bn, hdim), lambda i: (i, 0)),
        out_shape=jax.ShapeDtypeStruct((n, hdim), jnp.float32),
    )(x, w1.astype(jnp.bfloat16), b1.reshape(1, -1),
      w2.astype(jnp.bfloat16), b2.reshape(1, -1))


def _lane_extract(vec, j):
    mask = lax.iota(jnp.int32, _L) == j
    return jnp.max(jnp.where(mask, vec, jnp.int32(-1)))


def _searchsorted(ids_ref, targets, n_elems, steps):
    lo = jnp.zeros((_L,), jnp.int32)
    hi = jnp.full((_L,), n_elems, jnp.int32)
    for _ in range(steps):
        active = lo < hi
        mid = lax.shift_right_logical(lo + hi, 1)
        midc = jnp.minimum(mid, jnp.int32(n_elems - 1))
        v = plsc.load_gather(ids_ref, [midc])
        goright = active & (v < targets)
        lo = jnp.where(goright, mid + 1, lo)
        hi = jnp.where(active & jnp.logical_not(goright), mid, hi)
    return lo


def _pool_kernel_body(n_items, b_envs, hdim, h1d_hbm, idsub_hbm,
                      ids2d_hbm, out_hbm, cnt_hbm, idsub_v, idxb_v,
                      fine_v, buf0_v, buf1_v, out_v, cnt_v, sem, sem1):
    wid = lax.axis_index("s") * _NC + lax.axis_index("c")
    envs_per_w = b_envs // _NW
    base = wid * envs_per_w
    nk = hdim // _L
    nsub = n_items // _SUB
    csteps = nsub.bit_length()

    pltpu.async_copy(idsub_hbm, idsub_v, sem).wait()

    iot = lax.iota(jnp.int32, _L)
    nb = (envs_per_w + _L) // _L * _L
    rows = []
    for g in range(nb // _L):
        t = base + g * _L + iot
        cpos = _searchsorted(idsub_v, t, nsub, csteps)
        rows.append(jnp.maximum(cpos - 1, 0))
    for g in range(nb // _L):
        idxb_v[pl.ds(g * _L, _L)] = rows[g]
    pltpu.async_copy(ids2d_hbm.at[idxb_v], fine_v, sem).wait()

    starts = []
    for e in range(envs_per_w + 1):
        t = base + e
        cnt = jnp.int32(0)
        for k in range(_SUB // _L):
            v = fine_v[e, pl.ds(k * _L, _L)]
            cnt = cnt + jnp.max(
                plsc.all_reduce_population_count(v < t))
        row = _lane_extract(rows[e // _L], e % _L)
        starts.append(row * _SUB + cnt)

    cv = [jnp.zeros((_L,), jnp.float32) for _ in range(envs_per_w // _L)]
    zero = jnp.zeros((_L,), jnp.float32)
    for e in range(envs_per_w):
        cnt = starts[e + 1] - starts[e]
        cv[e // _L] = jnp.where(iot == (e % _L), cnt.astype(jnp.float32),
                                cv[e // _L])
        for k in range(2 * nk):
            out_v[e, pl.ds(k * _L, _L)] = zero

    lo = starts[0]
    hi = starts[envs_per_w]
    nch = lax.div(hi - lo + (_CHUNK - 1), jnp.int32(_CHUNK))
    npair = lax.div(nch + 1, jnp.int32(2))

    def reduce_chunk(clo, cs, buf):
        n = jnp.minimum(jnp.int32(_CHUNK), hi - clo)
        for e in range(envs_per_w):
            a = jnp.maximum(starts[e] - cs, clo - cs)
            b2 = jnp.minimum(starts[e + 1] - cs, (clo - cs) + n)

            @pl.when(b2 > a)
            def _(e=e, a=a, b2=b2, buf=buf):
                init = tuple(out_v[e, pl.ds(k * _L, _L)]
                             for k in range(2 * nk))

                def row_body(r, acc2):
                    rb = r * hdim
                    vs = [buf[pl.ds(rb + k * _L, _L)]
                          for k in range(nk)]
                    return (tuple(acc2[k] + vs[k] for k in range(nk))
                            + tuple(jnp.maximum(acc2[nk + k], vs[k])
                                    for k in range(nk)))

                acc = lax.fori_loop(a, b2, row_body, init)
                for k in range(2 * nk):
                    out_v[e, pl.ds(k * _L, _L)] = acc[k]

    def pair_body(i, _):
        clo0 = lo + (2 * i) * _CHUNK
        clo1 = clo0 + _CHUNK
        nmax = jnp.int32(n_items - _CHUNK)
        cs0 = jnp.minimum(clo0, nmax)
        cs1 = jnp.minimum(clo1, nmax)
        d0 = pltpu.async_copy(
            h1d_hbm.at[pl.ds(cs0 * hdim, _CHUNK * hdim)], buf0_v, sem)
        d1 = pltpu.async_copy(
            h1d_hbm.at[pl.ds(cs1 * hdim, _CHUNK * hdim)], buf1_v, sem1)
        d0.wait()
        reduce_chunk(clo0, cs0, buf0_v)
        d1.wait()
        reduce_chunk(clo1, cs1, buf1_v)
        return 0

    lax.fori_loop(0, npair, pair_body, 0)

    for j in range(envs_per_w // _L):
        cnt_v[pl.ds(j * _L, _L)] = cv[j]
    pltpu.sync_copy(out_v, out_hbm.at[pl.ds(base, envs_per_w)])
    pltpu.sync_copy(cnt_v, cnt_hbm.at[pl.ds(base, envs_per_w)])


def _sc_pool(h, ids):
    n, hdim = h.shape
    b = 1024
    h1d = h.reshape(-1)
    idsub = ids[::_SUB]
    ids2d = ids.reshape(n // _SUB, _SUB)
    envs_per_w = b // _NW
    nb = (envs_per_w + _L) // _L * _L
    body = functools.partial(_pool_kernel_body, n, b, hdim)
    return pl.kernel(
        body,
        out_type=[
            jax.ShapeDtypeStruct((b, 2 * hdim), jnp.float32),
            jax.ShapeDtypeStruct((b,), jnp.float32),
        ],
        mesh=_mesh(),
        scratch_types=[
            pltpu.VMEM((n // _SUB,), jnp.int32),
            pltpu.VMEM((nb,), jnp.int32),
            pltpu.VMEM((nb, _SUB), jnp.int32),
            pltpu.VMEM((_CHUNK * hdim,), jnp.float32),
            pltpu.VMEM((_CHUNK * hdim,), jnp.float32),
            pltpu.VMEM((envs_per_w, 2 * hdim), jnp.float32),
            pltpu.VMEM((envs_per_w,), jnp.float32),
            pltpu.SemaphoreType.DMA,
            pltpu.SemaphoreType.DMA,
        ],
        compiler_params=pltpu.CompilerParams(needs_layout_passes=False),
    )(h1d, idsub, ids2d)


def _env_body(sm_ref, cnt_ref, glob_ref, w1cp_ref, w1cg_ref, b1_ref,
              vw1p_ref, vw1g_ref, vb1_ref, vw2_ref, vb2_ref, vw_ref,
              vb_ref, p_ref, val_ref):
    hdim = sm_ref.shape[1] // 2
    seg_sum = sm_ref[:, :hdim]
    seg_max = sm_ref[:, hdim:]
    inv = 1.0 / jnp.maximum(cnt_ref[...], 1.0)
    pooled = jnp.concatenate([seg_sum, seg_sum * inv, seg_max], axis=1)
    glob = glob_ref[...]
    p_ref[...] = (
        jnp.dot(pooled, w1cp_ref[...], preferred_element_type=jnp.float32)
        + jnp.dot(glob, w1cg_ref[...], preferred_element_type=jnp.float32)
        + b1_ref[...])
    vh = jnp.maximum(
        jnp.dot(pooled, vw1p_ref[...], preferred_element_type=jnp.float32)
        + jnp.dot(glob, vw1g_ref[...], preferred_element_type=jnp.float32)
        + vb1_ref[...], 0.0)
    vh = jnp.maximum(
        jnp.dot(vh, vw2_ref[...], preferred_element_type=jnp.float32)
        + vb2_ref[...], 0.0)
    val_ref[...] = jnp.dot(vh, vw_ref[...],
                           preferred_element_type=jnp.float32) + vb_ref[...]


def _tc_env(summax, cnt, glob, w1c, core_b1, val_w1, val_b1, val_w2,
            val_b2, v_w, v_b):
    b = summax.shape[0]
    hdim = summax.shape[1] // 2
    threeh = 3 * hdim
    return pl.pallas_call(
        _env_body,
        out_shape=[
            jax.ShapeDtypeStruct((b, hdim), jnp.float32),
            jax.ShapeDtypeStruct((b, 1), jnp.float32),
        ],
    )(summax, cnt.reshape(b, 1), glob, w1c[:threeh], w1c[threeh:],
      core_b1.reshape(1, -1), val_w1[:threeh], val_w1[threeh:],
      val_b1.reshape(1, -1), val_w2, val_b2.reshape(1, -1), v_w,
      v_b.reshape(1, -1))


_NSLOT = 6


def _gather_body(n_items, hdim, p_hbm, ids_hbm, out_hbm, p_sh, idx_v,
                 rows_v, stage_sem, gsems, wsems):
    wid = lax.axis_index("s") * _NC + lax.axis_index("c")
    sid = lax.axis_index("s")
    per_w = n_items // _NW
    base = wid * per_w
    gchunk = 128
    nj = per_w // gchunk

    @pl.when(sid == 0)
    def _():
        pltpu.async_copy(p_hbm, p_sh, stage_sem).wait()

    pltpu.sync_copy(ids_hbm.at[pl.ds(base, per_w)], idx_v)
    plsc.subcore_barrier()

    def gather(j):
        return pltpu.async_copy(
            p_sh.at[idx_v.at[pl.ds(j * gchunk, gchunk)]],
            rows_v.at[j % _NSLOT], gsems[j % _NSLOT])

    def write(j):
        return pltpu.async_copy(
            rows_v.at[j % _NSLOT],
            out_hbm.at[pl.ds(base + j * gchunk, gchunk)],
            wsems[j % _NSLOT])

    g = {}
    w = {}
    lag = 2
    for j in range(nj):
        if j >= _NSLOT:
            w[j - _NSLOT].wait()
        g[j] = gather(j)
        if j >= lag:
            g[j - lag].wait()
            w[j - lag] = write(j - lag)
    for j in range(nj - lag, nj):
        g[j].wait()
        w[j] = write(j)
    for j in range(max(0, nj - _NSLOT), nj):
        w[j].wait()


def _sc_gather(p, ids):
    n = ids.shape[0]
    hdim = p.shape[1]
    b = p.shape[0]
    body = functools.partial(_gather_body, n, hdim)
    return pl.kernel(
        body,
        out_type=jax.ShapeDtypeStruct((n, hdim), jnp.float32),
        mesh=_mesh(),
        scratch_types=[
            pltpu.VMEM_SHARED((b, hdim), jnp.float32),
            pltpu.VMEM((n // _NW,), jnp.int32),
            pltpu.VMEM((_NSLOT, 128, hdim), jnp.float32),
            pltpu.SemaphoreType.DMA,
            [pltpu.SemaphoreType.DMA] * _NSLOT,
            [pltpu.SemaphoreType.DMA] * _NSLOT,
        ],
    )(p, ids)


def _core_body(h_ref, ctxp_ref, w1h_ref, w2_ref, b2_ref, muw_ref,
               mub_ref, lsw_ref, lsb_ref, mu_ref, ls_ref):
    pre1 = jnp.dot(h_ref[...].astype(jnp.bfloat16), w1h_ref[...],
                   preferred_element_type=jnp.float32)
    c = jnp.maximum(pre1 + ctxp_ref[...], 0.0)
    c = jnp.maximum(
        jnp.dot(c.astype(jnp.bfloat16), w2_ref[...],
                preferred_element_type=jnp.float32)
        + b2_ref[...], 0.0)
    mu_ref[...] = jnp.dot(c, muw_ref[...],
                          preferred_element_type=jnp.float32) + mub_ref[...]
    ls = jnp.dot(c, lsw_ref[...],
                 preferred_element_type=jnp.float32) + lsb_ref[...]
    ls_ref[...] = jnp.clip(ls, -5.0, 2.0)


def _tc_core(h, ctxp, w1h, core_w2, core_b2, mu_w, mu_b, ls_w, ls_b):
    n, hdim = h.shape
    bn = 2048
    grid = n // bn
    const = lambda i: (0, 0)
    return pl.pallas_call(
        _core_body,
        grid=(grid,),
        in_specs=[
            pl.BlockSpec((bn, hdim), lambda i: (i, 0)),
            pl.BlockSpec((bn, hdim), lambda i: (i, 0)),
            pl.BlockSpec((hdim, hdim), const),
            pl.BlockSpec((hdim, hdim), const),
            pl.BlockSpec((1, hdim), const),
            pl.BlockSpec((hdim, 1), const),
            pl.BlockSpec((1, 1), const),
            pl.BlockSpec((hdim, 1), const),
            pl.BlockSpec((1, 1), const),
        ],
        out_specs=[
            pl.BlockSpec((bn, 1), lambda i: (i, 0)),
            pl.BlockSpec((bn, 1), lambda i: (i, 0)),
        ],
        out_shape=[
            jax.ShapeDtypeStruct((n, 1), jnp.float32),
            jax.ShapeDtypeStruct((n, 1), jnp.float32),
        ],
    )(h, ctxp, w1h.astype(jnp.bfloat16), core_w2.astype(jnp.bfloat16),
      core_b2.reshape(1, -1), mu_w, mu_b.reshape(1, 1), ls_w,
      ls_b.reshape(1, 1))


def kernel(item_features, item_batch, global_features,
           enc_w1, enc_b1, enc_w2, enc_b2,
           core_w1, core_b1, core_w2, core_b2,
           mu_w, mu_b, ls_w, ls_b,
           val_w1, val_b1, val_w2, val_b2,
           v_w, v_b):
    ids = item_batch.astype(jnp.int32)
    hdim = enc_w1.shape[1]
    w1h = core_w1[:hdim]
    w1c = core_w1[hdim:]

    h = _tc_encode(item_features, enc_w1, enc_b1, enc_w2, enc_b2)
    summax, cnt = _sc_pool(h, ids)
    p_env, val2d = _tc_env(summax, cnt, global_features, w1c, core_b1,
                           val_w1, val_b1, val_w2, val_b2, v_w, v_b)
    ctxp = _sc_gather(p_env, ids)
    mu, log_std = _tc_core(h, ctxp, w1h, core_w2, core_b2, mu_w, mu_b,
                           ls_w, ls_b)
    return mu, log_std, val2d[:, 0]


# speedup vs baseline: 1.2408x; 1.2408x over previous
import functools

import jax
import jax.numpy as jnp
from jax import lax
from jax.experimental import pallas as pl
from jax.experimental.pallas import tpu as pltpu
from jax.experimental.pallas import tpu_sc as plsc

_NC = 2
_NS = 16
_NW = _NC * _NS
_L = 16
_CHUNK = 320
_SUB = 128


def _mesh():
    return plsc.VectorSubcoreMesh(
        core_axis_name="c", subcore_axis_name="s", num_cores=_NC,
        num_subcores=_NS)


def _enc_body(x_ref, w1_ref, b1_ref, w2_ref, b2_ref, h_ref):
    h = jnp.maximum(
        jnp.dot(x_ref[...], w1_ref[...],
                preferred_element_type=jnp.float32) + b1_ref[...], 0.0)
    h = jnp.maximum(
        jnp.dot(h, w2_ref[...],
                preferred_element_type=jnp.float32) + b2_ref[...], 0.0)
    h_ref[...] = h


def _tc_encode(x, w1, b1, w2, b2):
    n, din = x.shape
    hdim = w1.shape[1]
    bn = 2048
    grid = n // bn
    const = lambda i: (0, 0)
    return pl.pallas_call(
        _enc_body,
        grid=(grid,),
        in_specs=[
            pl.BlockSpec((bn, din), lambda i: (i, 0)),
            pl.BlockSpec((din, hdim), const),
            pl.BlockSpec((1, hdim), const),
            pl.BlockSpec((hdim, hdim), const),
            pl.BlockSpec((1, hdim), const),
        ],
        out_specs=pl.BlockSpec((bn, hdim), lambda i: (i, 0)),
        out_shape=jax.ShapeDtypeStruct((n, hdim), jnp.float32),
        compiler_params=pltpu.CompilerParams(
            allow_input_fusion=(True, False, False, False, False)),
    )(x, w1.astype(jnp.float32), b1.reshape(1, -1), w2,
      b2.reshape(1, -1))


def _lane_extract(vec, j):
    mask = lax.iota(jnp.int32, _L) == j
    return jnp.max(jnp.where(mask, vec, jnp.int32(-1)))


def _searchsorted(ids_ref, targets, n_elems, steps):
    lo = jnp.zeros((_L,), jnp.int32)
    hi = jnp.full((_L,), n_elems, jnp.int32)
    for _ in range(steps):
        active = lo < hi
        mid = lax.shift_right_logical(lo + hi, 1)
        midc = jnp.minimum(mid, jnp.int32(n_elems - 1))
        v = plsc.load_gather(ids_ref, [midc])
        goright = active & (v < targets)
        lo = jnp.where(goright, mid + 1, lo)
        hi = jnp.where(active & jnp.logical_not(goright), mid, hi)
    return lo


def _pool_kernel_body(n_items, b_envs, hdim, h1d_hbm, idsub_hbm,
                      ids2d_hbm, out_hbm, cnt_hbm, idsub_v, idxb_v,
                      fine_v, buf0_v, buf1_v, out_v, cnt_v, sem, sem1):
    wid = lax.axis_index("s") * _NC + lax.axis_index("c")
    envs_per_w = b_envs // _NW
    base = wid * envs_per_w
    nk = hdim // _L
    nsub = n_items // _SUB
    csteps = nsub.bit_length()

    pltpu.async_copy(idsub_hbm, idsub_v, sem).wait()

    iot = lax.iota(jnp.int32, _L)
    nb = (envs_per_w + _L) // _L * _L
    rows = []
    for g in range(nb // _L):
        t = base + g * _L + iot
        cpos = _searchsorted(idsub_v, t, nsub, csteps)
        rows.append(jnp.maximum(cpos - 1, 0))
    for g in range(nb // _L):
        idxb_v[pl.ds(g * _L, _L)] = rows[g]
    pltpu.async_copy(ids2d_hbm.at[idxb_v], fine_v, sem).wait()

    starts = []
    for e in range(envs_per_w + 1):
        t = base + e
        cnt = jnp.int32(0)
        for k in range(_SUB // _L):
            v = fine_v[e, pl.ds(k * _L, _L)]
            cnt = cnt + jnp.max(
                plsc.all_reduce_population_count(v < t))
        row = _lane_extract(rows[e // _L], e % _L)
        starts.append(row * _SUB + cnt)

    cv = [jnp.zeros((_L,), jnp.float32) for _ in range(envs_per_w // _L)]
    zero = jnp.zeros((_L,), jnp.float32)
    for e in range(envs_per_w):
        cnt = starts[e + 1] - starts[e]
        cv[e // _L] = jnp.where(iot == (e % _L), cnt.astype(jnp.float32),
                                cv[e // _L])
        for k in range(2 * nk):
            out_v[e, pl.ds(k * _L, _L)] = zero

    lo = starts[0]
    hi = starts[envs_per_w]
    nch = lax.div(hi - lo + (_CHUNK - 1), jnp.int32(_CHUNK))
    npair = lax.div(nch + 1, jnp.int32(2))

    def reduce_chunk(clo, cs, buf):
        n = jnp.minimum(jnp.int32(_CHUNK), hi - clo)
        for e in range(envs_per_w):
            a = jnp.maximum(starts[e] - cs, clo - cs)
            b2 = jnp.minimum(starts[e + 1] - cs, (clo - cs) + n)

            @pl.when(b2 > a)
            def _(e=e, a=a, b2=b2, buf=buf):
                init = tuple(out_v[e, pl.ds(k * _L, _L)]
                             for k in range(2 * nk))

                def row_body(r, acc2):
                    rb = r * hdim
                    vs = [buf[pl.ds(rb + k * _L, _L)]
                          for k in range(nk)]
                    return (tuple(acc2[k] + vs[k] for k in range(nk))
                            + tuple(jnp.maximum(acc2[nk + k], vs[k])
                                    for k in range(nk)))

                acc = lax.fori_loop(a, b2, row_body, init)
                for k in range(2 * nk):
                    out_v[e, pl.ds(k * _L, _L)] = acc[k]

    def pair_body(i, _):
        clo0 = lo + (2 * i) * _CHUNK
        clo1 = clo0 + _CHUNK
        nmax = jnp.int32(n_items - _CHUNK)
        cs0 = jnp.minimum(clo0, nmax)
        cs1 = jnp.minimum(clo1, nmax)
        d0 = pltpu.async_copy(
            h1d_hbm.at[pl.ds(cs0 * hdim, _CHUNK * hdim)], buf0_v, sem)
        d1 = pltpu.async_copy(
            h1d_hbm.at[pl.ds(cs1 * hdim, _CHUNK * hdim)], buf1_v, sem1)
        d0.wait()
        reduce_chunk(clo0, cs0, buf0_v)
        d1.wait()
        reduce_chunk(clo1, cs1, buf1_v)
        return 0

    lax.fori_loop(0, npair, pair_body, 0)

    for j in range(envs_per_w // _L):
        cnt_v[pl.ds(j * _L, _L)] = cv[j]
    pltpu.sync_copy(out_v, out_hbm.at[pl.ds(base, envs_per_w)])
    pltpu.sync_copy(cnt_v, cnt_hbm.at[pl.ds(base, envs_per_w)])


def _sc_pool(h, ids):
    n, hdim = h.shape
    b = 1024
    h1d = h.reshape(-1)
    idsub = ids[::_SUB]
    ids2d = ids.reshape(n // _SUB, _SUB)
    envs_per_w = b // _NW
    nb = (envs_per_w + _L) // _L * _L
    body = functools.partial(_pool_kernel_body, n, b, hdim)
    return pl.kernel(
        body,
        out_type=[
            jax.ShapeDtypeStruct((b, 2 * hdim), jnp.float32),
            jax.ShapeDtypeStruct((b,), jnp.float32),
        ],
        mesh=_mesh(),
        scratch_types=[
            pltpu.VMEM((n // _SUB,), jnp.int32),
            pltpu.VMEM((nb,), jnp.int32),
            pltpu.VMEM((nb, _SUB), jnp.int32),
            pltpu.VMEM((_CHUNK * hdim,), jnp.float32),
            pltpu.VMEM((_CHUNK * hdim,), jnp.float32),
            pltpu.VMEM((envs_per_w, 2 * hdim), jnp.float32),
            pltpu.VMEM((envs_per_w,), jnp.float32),
            pltpu.SemaphoreType.DMA,
            pltpu.SemaphoreType.DMA,
        ],
        compiler_params=pltpu.CompilerParams(needs_layout_passes=False),
    )(h1d, idsub, ids2d)


def _env_body(sm_ref, cnt_ref, glob_ref, w1cp_ref, w1cg_ref, b1_ref,
              vw1p_ref, vw1g_ref, vb1_ref, vw2_ref, vb2_ref, vw_ref,
              vb_ref, p_ref, val_ref):
    hdim = sm_ref.shape[1] // 2
    seg_sum = sm_ref[:, :hdim]
    seg_max = sm_ref[:, hdim:]
    inv = 1.0 / jnp.maximum(cnt_ref[...], 1.0)
    pooled = jnp.concatenate([seg_sum, seg_sum * inv, seg_max], axis=1)
    glob = glob_ref[...]
    p_ref[...] = (
        jnp.dot(pooled, w1cp_ref[...], preferred_element_type=jnp.float32)
        + jnp.dot(glob, w1cg_ref[...], preferred_element_type=jnp.float32)
        + b1_ref[...])
    vh = jnp.maximum(
        jnp.dot(pooled, vw1p_ref[...], preferred_element_type=jnp.float32)
        + jnp.dot(glob, vw1g_ref[...], preferred_element_type=jnp.float32)
        + vb1_ref[...], 0.0)
    vh = jnp.maximum(
        jnp.dot(vh, vw2_ref[...], preferred_element_type=jnp.float32)
        + vb2_ref[...], 0.0)
    val_ref[...] = jnp.dot(vh, vw_ref[...],
                           preferred_element_type=jnp.float32) + vb_ref[...]


def _tc_env(summax, cnt, glob, w1c, core_b1, val_w1, val_b1, val_w2,
            val_b2, v_w, v_b):
    b = summax.shape[0]
    hdim = summax.shape[1] // 2
    threeh = 3 * hdim
    return pl.pallas_call(
        _env_body,
        out_shape=[
            jax.ShapeDtypeStruct((b, hdim), jnp.float32),
            jax.ShapeDtypeStruct((b, 1), jnp.float32),
        ],
    )(summax, cnt.reshape(b, 1), glob, w1c[:threeh], w1c[threeh:],
      core_b1.reshape(1, -1), val_w1[:threeh], val_w1[threeh:],
      val_b1.reshape(1, -1), val_w2, val_b2.reshape(1, -1), v_w,
      v_b.reshape(1, -1))


_NSLOT = 6


def _gather_body(n_items, hdim, p_hbm, ids_hbm, out_hbm, p_sh, idx_v,
                 rows_v, stage_sem, gsems, wsems):
    wid = lax.axis_index("s") * _NC + lax.axis_index("c")
    sid = lax.axis_index("s")
    per_w = n_items // _NW
    base = wid * per_w
    gchunk = 128
    nj = per_w // gchunk

    @pl.when(sid == 0)
    def _():
        pltpu.async_copy(p_hbm, p_sh, stage_sem).wait()

    pltpu.sync_copy(ids_hbm.at[pl.ds(base, per_w)], idx_v)
    plsc.subcore_barrier()

    def gather(j):
        return pltpu.async_copy(
            p_sh.at[idx_v.at[pl.ds(j * gchunk, gchunk)]],
            rows_v.at[j % _NSLOT], gsems[j % _NSLOT])

    def write(j):
        return pltpu.async_copy(
            rows_v.at[j % _NSLOT],
            out_hbm.at[pl.ds(base + j * gchunk, gchunk)],
            wsems[j % _NSLOT])

    g = {}
    w = {}
    lag = 2
    for j in range(nj):
        if j >= _NSLOT:
            w[j - _NSLOT].wait()
        g[j] = gather(j)
        if j >= lag:
            g[j - lag].wait()
            w[j - lag] = write(j - lag)
    for j in range(nj - lag, nj):
        g[j].wait()
        w[j] = write(j)
    for j in range(max(0, nj - _NSLOT), nj):
        w[j].wait()


def _sc_gather(p, ids):
    n = ids.shape[0]
    hdim = p.shape[1]
    b = p.shape[0]
    body = functools.partial(_gather_body, n, hdim)
    return pl.kernel(
        body,
        out_type=jax.ShapeDtypeStruct((n, hdim), jnp.float32),
        mesh=_mesh(),
        scratch_types=[
            pltpu.VMEM_SHARED((b, hdim), jnp.float32),
            pltpu.VMEM((n // _NW,), jnp.int32),
            pltpu.VMEM((_NSLOT, 128, hdim), jnp.float32),
            pltpu.SemaphoreType.DMA,
            [pltpu.SemaphoreType.DMA] * _NSLOT,
            [pltpu.SemaphoreType.DMA] * _NSLOT,
        ],
    )(p, ids)


def _core_body(h_ref, ctxp_ref, w1h_ref, w2_ref, b2_ref, muw_ref,
               mub_ref, lsw_ref, lsb_ref, mu_ref, ls_ref):
    pre1 = jnp.dot(h_ref[...], w1h_ref[...],
                   preferred_element_type=jnp.float32)
    c = jnp.maximum(pre1 + ctxp_ref[...], 0.0)
    c = jnp.maximum(
        jnp.dot(c, w2_ref[...], preferred_element_type=jnp.float32)
        + b2_ref[...], 0.0)
    rows = mu_ref.shape[0]
    mu = jnp.dot(c, muw_ref[...],
                 preferred_element_type=jnp.float32) + mub_ref[...]
    mu_ref[...] = mu.reshape(rows, 128)
    ls = jnp.dot(c, lsw_ref[...],
                 preferred_element_type=jnp.float32) + lsb_ref[...]
    ls_ref[...] = jnp.clip(ls, -5.0, 2.0).reshape(rows, 128)


def _tc_core(h, ctxp, w1h, core_w2, core_b2, mu_w, mu_b, ls_w, ls_b):
    n, hdim = h.shape
    bn = 2048
    grid = n // bn
    const = lambda i: (0, 0)
    return pl.pallas_call(
        _core_body,
        grid=(grid,),
        in_specs=[
            pl.BlockSpec((bn, hdim), lambda i: (i, 0)),
            pl.BlockSpec((bn, hdim), lambda i: (i, 0)),
            pl.BlockSpec((hdim, hdim), const),
            pl.BlockSpec((hdim, hdim), const),
            pl.BlockSpec((1, hdim), const),
            pl.BlockSpec((hdim, 1), const),
            pl.BlockSpec((1, 1), const),
            pl.BlockSpec((hdim, 1), const),
            pl.BlockSpec((1, 1), const),
        ],
        out_specs=[
            pl.BlockSpec((bn // 128, 128), lambda i: (i, 0)),
            pl.BlockSpec((bn // 128, 128), lambda i: (i, 0)),
        ],
        out_shape=[
            jax.ShapeDtypeStruct((n // 128, 128), jnp.float32),
            jax.ShapeDtypeStruct((n // 128, 128), jnp.float32),
        ],
    )(h, ctxp, w1h, core_w2, core_b2.reshape(1, -1), mu_w,
      mu_b.reshape(1, 1), ls_w, ls_b.reshape(1, 1))


def kernel(item_features, item_batch, global_features,
           enc_w1, enc_b1, enc_w2, enc_b2,
           core_w1, core_b1, core_w2, core_b2,
           mu_w, mu_b, ls_w, ls_b,
           val_w1, val_b1, val_w2, val_b2,
           v_w, v_b):
    ids = item_batch.astype(jnp.int32)
    hdim = enc_w1.shape[1]
    w1h = core_w1[:hdim]
    w1c = core_w1[hdim:]

    h = _tc_encode(item_features, enc_w1, enc_b1, enc_w2, enc_b2)
    summax, cnt = _sc_pool(h, ids)
    p_env, val2d = _tc_env(summax, cnt, global_features, w1c, core_b1,
                           val_w1, val_b1, val_w2, val_b2, v_w, v_b)
    ctxp = _sc_gather(p_env, ids)
    mu, log_std = _tc_core(h, ctxp, w1h, core_w2, core_b2, mu_w, mu_b,
                           ls_w, ls_b)
    n = item_features.shape[0]
    return mu.reshape(n, 1), log_std.reshape(n, 1), val2d[:, 0]


# speedup vs baseline: 1.4281x; 1.1509x over previous
import functools

import jax
import jax.numpy as jnp
from jax import lax
from jax.experimental import pallas as pl
from jax.experimental.pallas import tpu as pltpu
from jax.experimental.pallas import tpu_sc as plsc

_NC = 2
_NS = 16
_NW = _NC * _NS
_L = 16
_CHUNK = 320
_SUB = 128


def _mesh():
    return plsc.VectorSubcoreMesh(
        core_axis_name="c", subcore_axis_name="s", num_cores=_NC,
        num_subcores=_NS)


def _enc_body(xt_ref, w1_ref, b1_ref, w2_ref, b2_ref, h_ref):
    h = jnp.maximum(
        lax.dot_general(xt_ref[...], w1_ref[...],
                        (((0,), (0,)), ((), ())),
                        preferred_element_type=jnp.float32)
        + b1_ref[...], 0.0)
    h = jnp.maximum(
        jnp.dot(h, w2_ref[...],
                preferred_element_type=jnp.float32) + b2_ref[...], 0.0)
    h_ref[...] = h


def _tc_encode(xt, w1, b1, w2, b2):
    din, n = xt.shape
    hdim = w1.shape[1]
    bn = 2048
    grid = n // bn
    const = lambda i: (0, 0)
    return pl.pallas_call(
        _enc_body,
        grid=(grid,),
        in_specs=[
            pl.BlockSpec((din, bn), lambda i: (0, i)),
            pl.BlockSpec((din, hdim), const),
            pl.BlockSpec((1, hdim), const),
            pl.BlockSpec((hdim, hdim), const),
            pl.BlockSpec((1, hdim), const),
        ],
        out_specs=pl.BlockSpec((bn, hdim), lambda i: (i, 0)),
        out_shape=jax.ShapeDtypeStruct((n, hdim), jnp.float32),
    )(xt, w1.astype(jnp.float32), b1.reshape(1, -1), w2,
      b2.reshape(1, -1))


def _lane_extract(vec, j):
    mask = lax.iota(jnp.int32, _L) == j
    return jnp.max(jnp.where(mask, vec, jnp.int32(-1)))


def _searchsorted(ids_ref, targets, n_elems, steps):
    lo = jnp.zeros((_L,), jnp.int32)
    hi = jnp.full((_L,), n_elems, jnp.int32)
    for _ in range(steps):
        active = lo < hi
        mid = lax.shift_right_logical(lo + hi, 1)
        midc = jnp.minimum(mid, jnp.int32(n_elems - 1))
        v = plsc.load_gather(ids_ref, [midc])
        goright = active & (v < targets)
        lo = jnp.where(goright, mid + 1, lo)
        hi = jnp.where(active & jnp.logical_not(goright), mid, hi)
    return lo


def _pool_kernel_body(n_items, b_envs, hdim, h1d_hbm, idsub_hbm,
                      ids2d_hbm, out_hbm, cnt_hbm, idsub_v, idxb_v,
                      fine_v, buf0_v, buf1_v, out_v, cnt_v, sem, sem1):
    wid = lax.axis_index("s") * _NC + lax.axis_index("c")
    envs_per_w = b_envs // _NW
    base = wid * envs_per_w
    nk = hdim // _L
    nsub = n_items // _SUB
    csteps = nsub.bit_length()

    pltpu.async_copy(idsub_hbm, idsub_v, sem).wait()

    iot = lax.iota(jnp.int32, _L)
    nb = (envs_per_w + _L) // _L * _L
    rows = []
    for g in range(nb // _L):
        t = base + g * _L + iot
        cpos = _searchsorted(idsub_v, t, nsub, csteps)
        rows.append(jnp.maximum(cpos - 1, 0))
    for g in range(nb // _L):
        idxb_v[pl.ds(g * _L, _L)] = rows[g]
    pltpu.async_copy(ids2d_hbm.at[idxb_v], fine_v, sem).wait()

    starts = []
    for e in range(envs_per_w + 1):
        t = base + e
        cnt = jnp.int32(0)
        for k in range(_SUB // _L):
            v = fine_v[e, pl.ds(k * _L, _L)]
            cnt = cnt + jnp.max(
                plsc.all_reduce_population_count(v < t))
        row = _lane_extract(rows[e // _L], e % _L)
        starts.append(row * _SUB + cnt)

    cv = [jnp.zeros((_L,), jnp.float32) for _ in range(envs_per_w // _L)]
    zero = jnp.zeros((_L,), jnp.float32)
    for e in range(envs_per_w):
        cnt = starts[e + 1] - starts[e]
        cv[e // _L] = jnp.where(iot == (e % _L), cnt.astype(jnp.float32),
                                cv[e // _L])
        for k in range(2 * nk):
            out_v[e, pl.ds(k * _L, _L)] = zero

    lo = starts[0]
    hi = starts[envs_per_w]
    nch = lax.div(hi - lo + (_CHUNK - 1), jnp.int32(_CHUNK))
    npair = lax.div(nch + 1, jnp.int32(2))

    def chunk_src(c):
        clo = lo + c * _CHUNK
        cs = jnp.minimum(clo, jnp.int32(n_items - _CHUNK))
        return h1d_hbm.at[pl.ds(cs * hdim, _CHUNK * hdim)], clo, cs

    def reduce_chunk(clo, cs, buf):
        n = jnp.minimum(jnp.int32(_CHUNK), hi - clo)
        for e in range(envs_per_w):
            a = jnp.maximum(starts[e] - cs, clo - cs)
            b2 = jnp.minimum(starts[e + 1] - cs, (clo - cs) + n)

            @pl.when(b2 > a)
            def _(e=e, a=a, b2=b2, buf=buf):
                init = tuple(out_v[e, pl.ds(k * _L, _L)]
                             for k in range(2 * nk))
                nrows = b2 - a

                def row2_body(j, acc2):
                    rb = (a + 2 * j) * hdim
                    vs = [buf[pl.ds(rb + k * _L, _L)]
                          for k in range(nk)]
                    ws = [buf[pl.ds(rb + hdim + k * _L, _L)]
                          for k in range(nk)]
                    return (tuple(acc2[k] + vs[k] + ws[k]
                                  for k in range(nk))
                            + tuple(jnp.maximum(acc2[nk + k],
                                                jnp.maximum(vs[k], ws[k]))
                                    for k in range(nk)))

                acc = lax.fori_loop(
                    0, lax.shift_right_logical(nrows, 1), row2_body,
                    init)
                fv = lax.broadcast_in_dim(
                    (nrows & 1).astype(jnp.float32), (_L,), ())
                rb = (b2 - 1) * hdim
                ts = [buf[pl.ds(rb + k * _L, _L)] * fv
                      for k in range(nk)]
                acc = (tuple(acc[k] + ts[k] for k in range(nk))
                       + tuple(jnp.maximum(acc[nk + k], ts[k])
                               for k in range(nk)))
                for k in range(2 * nk):
                    out_v[e, pl.ds(k * _L, _L)] = acc[k]

    src0, _, _ = chunk_src(0)
    pltpu.async_copy(src0, buf0_v, sem)
    src1, _, _ = chunk_src(1)
    pltpu.async_copy(src1, buf1_v, sem1)

    def pair_body(i, _):
        src0, clo0, cs0 = chunk_src(2 * i)
        pltpu.make_async_copy(src0, buf0_v, sem).wait()
        reduce_chunk(clo0, cs0, buf0_v)
        nsrc0, _, _ = chunk_src(2 * i + 2)
        pltpu.async_copy(nsrc0, buf0_v, sem)
        src1, clo1, cs1 = chunk_src(2 * i + 1)
        pltpu.make_async_copy(src1, buf1_v, sem1).wait()
        reduce_chunk(clo1, cs1, buf1_v)
        nsrc1, _, _ = chunk_src(2 * i + 3)
        pltpu.async_copy(nsrc1, buf1_v, sem1)
        return 0

    lax.fori_loop(0, npair, pair_body, 0)
    srcd, _, _ = chunk_src(0)
    pltpu.make_async_copy(srcd, buf0_v, sem).wait()
    pltpu.make_async_copy(srcd, buf1_v, sem1).wait()

    for j in range(envs_per_w // _L):
        cnt_v[pl.ds(j * _L, _L)] = cv[j]
    pltpu.sync_copy(out_v, out_hbm.at[pl.ds(base, envs_per_w)])
    pltpu.sync_copy(cnt_v, cnt_hbm.at[pl.ds(base, envs_per_w)])


def _sc_pool(h, ids):
    n, hdim = h.shape
    b = 1024
    h1d = h.reshape(-1)
    idsub = ids[::_SUB]
    ids2d = ids.reshape(n // _SUB, _SUB)
    envs_per_w = b // _NW
    nb = (envs_per_w + _L) // _L * _L
    body = functools.partial(_pool_kernel_body, n, b, hdim)
    return pl.kernel(
        body,
        out_type=[
            jax.ShapeDtypeStruct((b, 2 * hdim), jnp.float32),
            jax.ShapeDtypeStruct((b,), jnp.float32),
        ],
        mesh=_mesh(),
        scratch_types=[
            pltpu.VMEM((n // _SUB,), jnp.int32),
            pltpu.VMEM((nb,), jnp.int32),
            pltpu.VMEM((nb, _SUB), jnp.int32),
            pltpu.VMEM((_CHUNK * hdim,), jnp.float32),
            pltpu.VMEM((_CHUNK * hdim,), jnp.float32),
            pltpu.VMEM((envs_per_w, 2 * hdim), jnp.float32),
            pltpu.VMEM((envs_per_w,), jnp.float32),
            pltpu.SemaphoreType.DMA,
            pltpu.SemaphoreType.DMA,
        ],
        compiler_params=pltpu.CompilerParams(needs_layout_passes=False),
    )(h1d, idsub, ids2d)


def _env_body(sm_ref, cnt_ref, glob_ref, w1cp_ref, w1cg_ref, b1_ref,
              vw1p_ref, vw1g_ref, vb1_ref, vw2_ref, vb2_ref, vw_ref,
              vb_ref, p_ref, val_ref):
    hdim = sm_ref.shape[1] // 2
    seg_sum = sm_ref[:, :hdim]
    seg_max = sm_ref[:, hdim:]
    inv = 1.0 / jnp.maximum(cnt_ref[...], 1.0)
    pooled = jnp.concatenate([seg_sum, seg_sum * inv, seg_max], axis=1)
    glob = glob_ref[...]
    p_ref[...] = (
        jnp.dot(pooled, w1cp_ref[...], preferred_element_type=jnp.float32)
        + jnp.dot(glob, w1cg_ref[...], preferred_element_type=jnp.float32)
        + b1_ref[...])
    vh = jnp.maximum(
        jnp.dot(pooled, vw1p_ref[...], preferred_element_type=jnp.float32)
        + jnp.dot(glob, vw1g_ref[...], preferred_element_type=jnp.float32)
        + vb1_ref[...], 0.0)
    vh = jnp.maximum(
        jnp.dot(vh, vw2_ref[...], preferred_element_type=jnp.float32)
        + vb2_ref[...], 0.0)
    val_ref[...] = jnp.dot(vh, vw_ref[...],
                           preferred_element_type=jnp.float32) + vb_ref[...]


def _tc_env(summax, cnt, glob, w1c, core_b1, val_w1, val_b1, val_w2,
            val_b2, v_w, v_b):
    b = summax.shape[0]
    hdim = summax.shape[1] // 2
    threeh = 3 * hdim
    return pl.pallas_call(
        _env_body,
        out_shape=[
            jax.ShapeDtypeStruct((b, hdim), jnp.float32),
            jax.ShapeDtypeStruct((b, 1), jnp.float32),
        ],
    )(summax, cnt.reshape(b, 1), glob, w1c[:threeh], w1c[threeh:],
      core_b1.reshape(1, -1), val_w1[:threeh], val_w1[threeh:],
      val_b1.reshape(1, -1), val_w2, val_b2.reshape(1, -1), v_w,
      v_b.reshape(1, -1))


_NSLOT = 6


def _gather_body(n_items, hdim, p_hbm, ids_hbm, out_hbm, p_sh, idx_v,
                 rows_v, stage_sem, gsems, wsems):
    wid = lax.axis_index("s") * _NC + lax.axis_index("c")
    sid = lax.axis_index("s")
    per_w = n_items // _NW
    base = wid * per_w
    gchunk = 128
    nj = per_w // gchunk

    @pl.when(sid == 0)
    def _():
        pltpu.async_copy(p_hbm, p_sh, stage_sem).wait()

    pltpu.sync_copy(ids_hbm.at[pl.ds(base, per_w)], idx_v)
    plsc.subcore_barrier()

    def gather(j):
        return pltpu.async_copy(
            p_sh.at[idx_v.at[pl.ds(j * gchunk, gchunk)]],
            rows_v.at[j % _NSLOT], gsems[j % _NSLOT])

    def write(j):
        return pltpu.async_copy(
            rows_v.at[j % _NSLOT],
            out_hbm.at[pl.ds(base + j * gchunk, gchunk)],
            wsems[j % _NSLOT])

    g = {}
    w = {}
    lag = 2
    for j in range(nj):
        if j >= _NSLOT:
            w[j - _NSLOT].wait()
        g[j] = gather(j)
        if j >= lag:
            g[j - lag].wait()
            w[j - lag] = write(j - lag)
    for j in range(nj - lag, nj):
        g[j].wait()
        w[j] = write(j)
    for j in range(max(0, nj - _NSLOT), nj):
        w[j].wait()


def _sc_gather(p, ids):
    n = ids.shape[0]
    hdim = p.shape[1]
    b = p.shape[0]
    body = functools.partial(_gather_body, n, hdim)
    return pl.kernel(
        body,
        out_type=jax.ShapeDtypeStruct((n, hdim), jnp.float32),
        mesh=_mesh(),
        scratch_types=[
            pltpu.VMEM_SHARED((b, hdim), jnp.float32),
            pltpu.VMEM((n // _NW,), jnp.int32),
            pltpu.VMEM((_NSLOT, 128, hdim), jnp.float32),
            pltpu.SemaphoreType.DMA,
            [pltpu.SemaphoreType.DMA] * _NSLOT,
            [pltpu.SemaphoreType.DMA] * _NSLOT,
        ],
    )(p, ids)


def _core_body(h_ref, ctxp_ref, w1h_ref, w2_ref, b2_ref, muw_ref,
               mub_ref, lsw_ref, lsb_ref, mu_ref, ls_ref):
    pre1 = jnp.dot(h_ref[...], w1h_ref[...],
                   preferred_element_type=jnp.float32)
    c = jnp.maximum(pre1 + ctxp_ref[...], 0.0)
    c = jnp.maximum(
        jnp.dot(c, w2_ref[...], preferred_element_type=jnp.float32)
        + b2_ref[...], 0.0)
    rows = mu_ref.shape[0]
    mu = jnp.dot(c, muw_ref[...],
                 preferred_element_type=jnp.float32) + mub_ref[...]
    mu_ref[...] = mu.reshape(rows, 128)
    ls = jnp.dot(c, lsw_ref[...],
                 preferred_element_type=jnp.float32) + lsb_ref[...]
    ls_ref[...] = jnp.clip(ls, -5.0, 2.0).reshape(rows, 128)


def _tc_core(h, ctxp, w1h, core_w2, core_b2, mu_w, mu_b, ls_w, ls_b):
    n, hdim = h.shape
    bn = 2048
    grid = n // bn
    const = lambda i: (0, 0)
    return pl.pallas_call(
        _core_body,
        grid=(grid,),
        in_specs=[
            pl.BlockSpec((bn, hdim), lambda i: (i, 0)),
            pl.BlockSpec((bn, hdim), lambda i: (i, 0)),
            pl.BlockSpec((hdim, hdim), const),
            pl.BlockSpec((hdim, hdim), const),
            pl.BlockSpec((1, hdim), const),
            pl.BlockSpec((hdim, 1), const),
            pl.BlockSpec((1, 1), const),
            pl.BlockSpec((hdim, 1), const),
            pl.BlockSpec((1, 1), const),
        ],
        out_specs=[
            pl.BlockSpec((bn // 128, 128), lambda i: (i, 0)),
            pl.BlockSpec((bn // 128, 128), lambda i: (i, 0)),
        ],
        out_shape=[
            jax.ShapeDtypeStruct((n // 128, 128), jnp.float32),
            jax.ShapeDtypeStruct((n // 128, 128), jnp.float32),
        ],
    )(h, ctxp, w1h, core_w2, core_b2.reshape(1, -1), mu_w,
      mu_b.reshape(1, 1), ls_w, ls_b.reshape(1, 1))


def kernel(item_features, item_batch, global_features,
           enc_w1, enc_b1, enc_w2, enc_b2,
           core_w1, core_b1, core_w2, core_b2,
           mu_w, mu_b, ls_w, ls_b,
           val_w1, val_b1, val_w2, val_b2,
           v_w, v_b):
    ids = item_batch.astype(jnp.int32)
    hdim = enc_w1.shape[1]
    w1h = core_w1[:hdim]
    w1c = core_w1[hdim:]

    h = _tc_encode(item_features.T, enc_w1, enc_b1, enc_w2, enc_b2)
    summax, cnt = _sc_pool(h, ids)
    p_env, val2d = _tc_env(summax, cnt, global_features, w1c, core_b1,
                           val_w1, val_b1, val_w2, val_b2, v_w, v_b)
    ctxp = _sc_gather(p_env, ids)
    mu, log_std = _tc_core(h, ctxp, w1h, core_w2, core_b2, mu_w, mu_b,
                           ls_w, ls_b)
    n = item_features.shape[0]
    return mu.reshape(n, 1), log_std.reshape(n, 1), val2d[:, 0]
